# baseline (speedup 1.0000x reference)
import jax
import jax.numpy as jnp
from jax import lax
from jax.experimental import pallas as pl
from jax.experimental.pallas import tpu as pltpu, tpu_sc as plsc

N_ROWS = 100000
D = 256
B = 16384
MOM = 0.2

NC = 2
NS = 16
NW = NC * NS
R = 3136
C = 96
CAP = ((R + C - 1) // C) * C
NCH = CAP // C
DV = D // 16

_SENT = 0x7FFFFFFF


def _take(v, idx):
  return jnp.take_along_axis(v, idx, axis=0)


def _body_ab(plab_hbm, fout_hbm, feat_hbm, dstl_hbm, nch_hbm, norm_hbm,
             labels_v, claim, srcs, dstl, nch_v, fbuf0, gbuf0, fbuf1, gbuf1,
             sem_g0, sem_g1, sem_s0, sem_s1):
  wid = lax.axis_index("s") * NC + lax.axis_index("c")
  lo = wid * R
  hi = lo + R
  iota = lax.iota(jnp.int32, 16)
  nxt_idx = (iota + 1) & 15

  pltpu.sync_copy(plab_hbm, labels_v)

  minus1 = jnp.full((16,), -1, jnp.int32)

  @pl.loop(0, R // 16)
  def _(k):
    claim[pl.ds(k * 16, 16)] = minus1

  @pl.loop(0, B // 16, unroll=2)
  def _(s):
    l = labels_v[pl.ds(s * 16, 16)]
    i = s * 16 + iota
    inr = (l >= lo) & (l < hi)
    comp = jnp.where(inr, (l << 14) | i, _SENT)
    sk, _ = plsc.sort_key_val(comp, comp)
    slab = sk >> 14
    nlab = _take(slab, nxt_idx)
    win = ((slab != nlab) | (iota == 15)) & (sk != _SENT)
    idx = jnp.where(win, slab - lo, 0)
    plsc.store_scatter(claim, (idx,), sk & 0x3FFF, mask=win)

  @pl.loop(0, R // 16, init_carry=jnp.int32(0))
  def count(k, cnt):
    c = claim[pl.ds(k * 16, 16)]
    m = c >= 0
    mi = jnp.where(m, jnp.int32(1), jnp.int32(0))
    cum = plsc.cumsum(mi)
    posw = jnp.where(m, cnt + cum - 1, 0)
    plsc.store_scatter(srcs, (posw,), c, mask=m)
    plsc.store_scatter(dstl, (posw,), lo + k * 16 + iota, mask=m)
    return cnt + jnp.sum(mi)

  k_cnt = count

  @pl.when(k_cnt > 0)
  def _():
    kpad = ((k_cnt + C - 1) // C) * C

    @pl.loop(0, C // 16)
    def _(j):
      offs = k_cnt + j * 16 + iota
      mk = offs < kpad
      offw = jnp.where(mk, offs, 0)
      srcoff = jnp.maximum(offw - C, 0)
      plsc.store_scatter(srcs, (offw,), plsc.load_gather(srcs, (srcoff,)),
                         mask=mk)
      plsc.store_scatter(dstl, (offw,), plsc.load_gather(dstl, (srcoff,)),
                         mask=mk)

  nchunks = (k_cnt + C - 1) // C
  nch_v[pl.ds(0, 16)] = jnp.full((16,), 0, jnp.int32) + nchunks

  pltpu.sync_copy(dstl.at[pl.ds(0, CAP)], dstl_hbm.at[pl.ds(wid * CAP, CAP)])
  pltpu.sync_copy(nch_v, nch_hbm.at[pl.ds(wid * 16, 16)])

  bufs = ((fbuf0, gbuf0, sem_g0, sem_s0), (fbuf1, gbuf1, sem_g1, sem_s1))

  def start_gather(t, fb, gb, sg):
    pltpu.make_async_copy(
        fout_hbm.at[srcs.at[pl.ds(t * C, C)]], fb, sg).start()
    pltpu.make_async_copy(
        feat_hbm.at[dstl.at[pl.ds(t * C, C)]], gb, sg).start()

  def wait_gather(t, fb, gb, sg):
    pltpu.make_async_copy(
        fout_hbm.at[srcs.at[pl.ds(t * C, C)]], fb, sg).wait()
    pltpu.make_async_copy(
        feat_hbm.at[dstl.at[pl.ds(t * C, C)]], gb, sg).wait()

  def norm_slice(t):
    return norm_hbm.at[pl.ds(wid * CAP + t * C, C)]

  @pl.when(nchunks > 0)
  def _():
    start_gather(0, fbuf0, gbuf0, sem_g0)

  @pl.loop(0, nchunks)
  def _(t):
    par = t & 1

    for p in range(2):
      fb, gb, sg, ss = bufs[p]

      @pl.when(par == p)
      def _():
        @pl.when(t >= 1)
        def _():
          ofb = bufs[1 - p][0]
          oss = bufs[1 - p][3]
          pltpu.make_async_copy(ofb, norm_slice(t - 1), oss).wait()

        @pl.when(t + 1 < nchunks)
        def _():
          nfb, ngb, nsg, _ = bufs[1 - p]
          start_gather(t + 1, nfb, ngb, nsg)

        wait_gather(t, fb, gb, sg)

        @pl.loop(0, C, unroll=2)
        def _(r):
          acc = jnp.zeros((16,), jnp.float32)
          m = []
          for j in range(DV):
            g = gb[r, pl.ds(j * 16, 16)]
            f = fb[r, pl.ds(j * 16, 16)]
            mj = MOM * g + (1.0 - MOM) * f
            m.append(mj)
            acc = acc + mj * mj
          tot = _take(plsc.cumsum(acc), jnp.full((16,), 15, jnp.int32))
          bits = plsc.bitcast(tot, jnp.int32)
          y = plsc.bitcast(jnp.int32(0x5F3759DF) - (bits >> 1), jnp.float32)
          for _ in range(2):
            y = y * (1.5 - 0.5 * tot * y * y)
          for j in range(DV):
            fb[r, pl.ds(j * 16, 16)] = m[j] * y

        pltpu.make_async_copy(fb, norm_slice(t), ss).start()

  @pl.when(nchunks > 0)
  def _():
    for p in range(2):
      fb, _, _, ss = bufs[p]

      @pl.when((nchunks - 1) & 1 == p)
      def _():
        pltpu.make_async_copy(fb, norm_slice(nchunks - 1), ss).wait()


def _body_b2(dstl_hbm, nch_hbm, norm_hbm, out_ref,
             dstl, dstl3d, nch_v, buf0, buf1,
             sem_l0, sem_l1, sem_s0, sem_s1):
  wid = lax.axis_index("s") * NC + lax.axis_index("c")

  pltpu.sync_copy(dstl_hbm.at[pl.ds(wid * CAP, CAP)], dstl.at[pl.ds(0, CAP)])
  pltpu.sync_copy(nch_hbm.at[pl.ds(wid * 16, 16)], nch_v)
  nchunks = nch_v[pl.ds(0, 16)][0]

  @pl.loop(0, CAP // 16)
  def _(k):
    v = dstl[pl.ds(k * 16, 16)]
    ch = k // (C // 16)
    off = (k - ch * (C // 16)) * 16
    dstl3d[ch, 0, pl.ds(off, 16)] = v

  bufs = ((buf0, sem_l0, sem_s0), (buf1, sem_l1, sem_s1))

  def norm_slice(t):
    return norm_hbm.at[pl.ds(wid * CAP + t * C, C)]

  @pl.when(nchunks > 0)
  def _():
    pltpu.make_async_copy(norm_slice(0), buf0, sem_l0).start()

  @pl.loop(0, nchunks)
  def _(t):
    par = t & 1

    for p in range(2):
      bf, sl, ss = bufs[p]

      @pl.when(par == p)
      def _():
        @pl.when(t >= 1)
        def _():
          obf, _, oss = bufs[1 - p]
          pltpu.make_async_copy(
              obf, out_ref.at[dstl3d.at[t - 1, 0]], oss).wait()

        @pl.when(t + 1 < nchunks)
        def _():
          nbf, nsl, _ = bufs[1 - p]
          pltpu.make_async_copy(norm_slice(t + 1), nbf, nsl).start()

        pltpu.make_async_copy(norm_slice(t), bf, sl).wait()
        pltpu.make_async_copy(bf, out_ref.at[dstl3d.at[t, 0]], ss).start()

  @pl.when(nchunks > 0)
  def _():
    for p in range(2):
      bf, _, ss = bufs[p]

      @pl.when((nchunks - 1) & 1 == p)
      def _():
        pltpu.make_async_copy(
            bf, out_ref.at[dstl3d.at[nchunks - 1, 0]], ss).wait()


def _body_copy(feat_hbm, out_ref, sem):
  nsp = 10
  rows = N_ROWS // nsp
  cps = [
      pltpu.make_async_copy(
          feat_hbm.at[pl.ds(k * rows, rows)],
          out_ref.at[pl.ds(k * rows, rows)], sem)
      for k in range(nsp)
  ]
  for c in cps:
    c.start()
  for c in cps:
    c.wait()


def kernel(f_out, p_labels, features):
  mesh = plsc.VectorSubcoreMesh(
      core_axis_name="c", subcore_axis_name="s", num_cores=NC)
  cp = pltpu.CompilerParams(needs_layout_passes=False)

  run_ab = pl.kernel(
      _body_ab,
      out_type=(
          jax.ShapeDtypeStruct((NW * CAP,), jnp.int32),
          jax.ShapeDtypeStruct((NW * 16,), jnp.int32),
          jax.ShapeDtypeStruct((NW * CAP, D), jnp.float32),
      ),
      mesh=mesh,
      compiler_params=cp,
      scratch_types=[
          pltpu.VMEM((B,), jnp.int32),
          pltpu.VMEM((R,), jnp.int32),
          pltpu.VMEM((CAP + 16,), jnp.int32),
          pltpu.VMEM((CAP + 16,), jnp.int32),
          pltpu.VMEM((16,), jnp.int32),
          pltpu.VMEM((C, D), jnp.float32),
          pltpu.VMEM((C, D), jnp.float32),
          pltpu.VMEM((C, D), jnp.float32),
          pltpu.VMEM((C, D), jnp.float32),
          pltpu.SemaphoreType.DMA,
          pltpu.SemaphoreType.DMA,
          pltpu.SemaphoreType.DMA,
          pltpu.SemaphoreType.DMA,
      ],
  )
  dstl_hbm, nch_hbm, norm_hbm = run_ab(p_labels, f_out, features)

  out_ref = jax.new_ref(lax.empty((N_ROWS, D), jnp.float32))
  run_copy = pl.kernel(
      _body_copy,
      out_type=(),
      mesh=pltpu.create_tensorcore_mesh("x", num_cores=1),
      scratch_types=[pltpu.SemaphoreType.DMA],
  )
  run_copy(features, out_ref)
  run_b2 = pl.kernel(
      _body_b2,
      out_type=(),
      mesh=mesh,
      compiler_params=cp,
      scratch_types=[
          pltpu.VMEM((CAP + 16,), jnp.int32),
          pltpu.VMEM((NCH, 1, C), jnp.int32),
          pltpu.VMEM((16,), jnp.int32),
          pltpu.VMEM((C, D), jnp.float32),
          pltpu.VMEM((C, D), jnp.float32),
          pltpu.SemaphoreType.DMA,
          pltpu.SemaphoreType.DMA,
          pltpu.SemaphoreType.DMA,
          pltpu.SemaphoreType.DMA,
      ],
  )
  run_b2(dstl_hbm, nch_hbm, norm_hbm, out_ref)
  return out_ref[...]


# speedup vs baseline: 25.2132x; 25.2132x over previous
import jax
import jax.numpy as jnp
from jax import lax
from jax.experimental import pallas as pl
from jax.experimental.pallas import tpu as pltpu, tpu_sc as plsc

N_ROWS = 100000
D = 256
B = 16384
MOM = 0.2

NC = 2
NS = 16
NW = NC * NS
R = 3136
C = 96
CAP = ((R + C - 1) // C) * C
NCH = CAP // C
DV = D // 16

_SENT = 0x7FFFFFFF


def _take(v, idx):
  return jnp.take_along_axis(v, idx, axis=0)


def _body_ab(plab_hbm, fout_hbm, feat_hbm, dstl_hbm, nch_hbm, norm_hbm,
             labels_v, claim, srcs, dstl, nch_v, fbuf0, gbuf0, fbuf1, gbuf1,
             sem_g0, sem_g1, sem_s0, sem_s1):
  wid = lax.axis_index("s") * NC + lax.axis_index("c")
  lo = wid * R
  hi = lo + R
  iota = lax.iota(jnp.int32, 16)
  nxt_idx = (iota + 1) & 15

  pltpu.sync_copy(plab_hbm, labels_v)

  minus1 = jnp.full((16,), -1, jnp.int32)

  @pl.loop(0, R // 16)
  def _(k):
    claim[pl.ds(k * 16, 16)] = minus1

  @pl.loop(0, B // 16, unroll=2)
  def _(s):
    l = labels_v[pl.ds(s * 16, 16)]
    i = s * 16 + iota
    inr = (l >= lo) & (l < hi)
    comp = jnp.where(inr, (l << 14) | i, _SENT)
    sk, _ = plsc.sort_key_val(comp, comp)
    slab = sk >> 14
    nlab = _take(slab, nxt_idx)
    win = ((slab != nlab) | (iota == 15)) & (sk != _SENT)
    idx = jnp.where(win, slab - lo, 0)
    plsc.store_scatter(claim, (idx,), sk & 0x3FFF, mask=win)

  @pl.loop(0, R // 16, init_carry=jnp.int32(0))
  def count(k, cnt):
    c = claim[pl.ds(k * 16, 16)]
    m = c >= 0
    mi = jnp.where(m, jnp.int32(1), jnp.int32(0))
    cum = plsc.cumsum(mi)
    posw = jnp.where(m, cnt + cum - 1, 0)
    plsc.store_scatter(srcs, (posw,), c, mask=m)
    plsc.store_scatter(dstl, (posw,), lo + k * 16 + iota, mask=m)
    return cnt + jnp.sum(mi)

  k_cnt = count

  @pl.when(k_cnt > 0)
  def _():
    kpad = ((k_cnt + C - 1) // C) * C

    @pl.loop(0, C // 16)
    def _(j):
      offs = k_cnt + j * 16 + iota
      mk = offs < kpad
      offw = jnp.where(mk, offs, 0)
      srcoff = jnp.maximum(offw - C, 0)
      plsc.store_scatter(srcs, (offw,), plsc.load_gather(srcs, (srcoff,)),
                         mask=mk)
      plsc.store_scatter(dstl, (offw,), plsc.load_gather(dstl, (srcoff,)),
                         mask=mk)

  nchunks = (k_cnt + C - 1) // C
  nch_v[pl.ds(0, 16)] = jnp.full((16,), 0, jnp.int32) + nchunks

  pltpu.sync_copy(dstl.at[pl.ds(0, CAP)], dstl_hbm.at[pl.ds(wid * CAP, CAP)])
  pltpu.sync_copy(nch_v, nch_hbm.at[pl.ds(wid * 16, 16)])

  bufs = ((fbuf0, gbuf0, sem_g0, sem_s0), (fbuf1, gbuf1, sem_g1, sem_s1))

  def start_gather(t, fb, gb, sg):
    pltpu.make_async_copy(
        fout_hbm.at[srcs.at[pl.ds(t * C, C)]], fb, sg).start()
    pltpu.make_async_copy(
        feat_hbm.at[dstl.at[pl.ds(t * C, C)]], gb, sg).start()

  def wait_gather(t, fb, gb, sg):
    pltpu.make_async_copy(
        fout_hbm.at[srcs.at[pl.ds(t * C, C)]], fb, sg).wait()
    pltpu.make_async_copy(
        feat_hbm.at[dstl.at[pl.ds(t * C, C)]], gb, sg).wait()

  def norm_slice(t):
    return norm_hbm.at[pl.ds(wid * CAP + t * C, C)]

  @pl.when(nchunks > 0)
  def _():
    start_gather(0, fbuf0, gbuf0, sem_g0)

  @pl.loop(0, nchunks)
  def _(t):
    par = t & 1

    for p in range(2):
      fb, gb, sg, ss = bufs[p]

      @pl.when(par == p)
      def _():
        @pl.when(t >= 1)
        def _():
          ofb = bufs[1 - p][0]
          oss = bufs[1 - p][3]
          pltpu.make_async_copy(ofb, norm_slice(t - 1), oss).wait()

        @pl.when(t + 1 < nchunks)
        def _():
          nfb, ngb, nsg, _ = bufs[1 - p]
          start_gather(t + 1, nfb, ngb, nsg)

        wait_gather(t, fb, gb, sg)

        @pl.loop(0, C, unroll=2)
        def _(r):
          acc = jnp.zeros((16,), jnp.float32)
          m = []
          for j in range(DV):
            g = gb[r, pl.ds(j * 16, 16)]
            f = fb[r, pl.ds(j * 16, 16)]
            mj = MOM * g + (1.0 - MOM) * f
            m.append(mj)
            acc = acc + mj * mj
          tot = _take(plsc.cumsum(acc), jnp.full((16,), 15, jnp.int32))
          bits = plsc.bitcast(tot, jnp.int32)
          y = plsc.bitcast(jnp.int32(0x5F3759DF) - (bits >> 1), jnp.float32)
          for _ in range(2):
            y = y * (1.5 - 0.5 * tot * y * y)
          for j in range(DV):
            fb[r, pl.ds(j * 16, 16)] = m[j] * y

        pltpu.make_async_copy(fb, norm_slice(t), ss).start()

  @pl.when(nchunks > 0)
  def _():
    for p in range(2):
      fb, _, _, ss = bufs[p]

      @pl.when((nchunks - 1) & 1 == p)
      def _():
        pltpu.make_async_copy(fb, norm_slice(nchunks - 1), ss).wait()


def _body_b2(dstl_hbm, nch_hbm, norm_hbm, out_ref,
             dstl, dstl3d, nch_v, buf0, buf1,
             sem_l0, sem_l1, sem_s0, sem_s1):
  wid = lax.axis_index("s") * NC + lax.axis_index("c")

  pltpu.sync_copy(dstl_hbm.at[pl.ds(wid * CAP, CAP)], dstl.at[pl.ds(0, CAP)])
  pltpu.sync_copy(nch_hbm.at[pl.ds(wid * 16, 16)], nch_v)
  nchunks = nch_v[pl.ds(0, 16)][0]

  @pl.loop(0, CAP // 16)
  def _(k):
    v = dstl[pl.ds(k * 16, 16)]
    ch = k // (C // 16)
    off = (k - ch * (C // 16)) * 16
    dstl3d[ch, 0, pl.ds(off, 16)] = v

  bufs = ((buf0, sem_l0, sem_s0), (buf1, sem_l1, sem_s1))

  def norm_slice(t):
    return norm_hbm.at[pl.ds(wid * CAP + t * C, C)]

  @pl.when(nchunks > 0)
  def _():
    pltpu.make_async_copy(norm_slice(0), buf0, sem_l0).start()

  @pl.loop(0, nchunks)
  def _(t):
    par = t & 1

    for p in range(2):
      bf, sl, ss = bufs[p]

      @pl.when(par == p)
      def _():
        @pl.when(t >= 1)
        def _():
          obf, _, oss = bufs[1 - p]
          pltpu.make_async_copy(
              obf, out_ref.at[dstl3d.at[t - 1, 0]], oss).wait()

        @pl.when(t + 1 < nchunks)
        def _():
          nbf, nsl, _ = bufs[1 - p]
          pltpu.make_async_copy(norm_slice(t + 1), nbf, nsl).start()

        pltpu.make_async_copy(norm_slice(t), bf, sl).wait()
        pltpu.make_async_copy(bf, out_ref.at[dstl3d.at[t, 0]], ss).start()

  @pl.when(nchunks > 0)
  def _():
    for p in range(2):
      bf, _, ss = bufs[p]

      @pl.when((nchunks - 1) & 1 == p)
      def _():
        pltpu.make_async_copy(
            bf, out_ref.at[dstl3d.at[nchunks - 1, 0]], ss).wait()


_COPY_BLK = 2000


def _copy_block(x_ref, o_ref):
  o_ref[...] = x_ref[...]


def kernel(f_out, p_labels, features):
  mesh = plsc.VectorSubcoreMesh(
      core_axis_name="c", subcore_axis_name="s", num_cores=NC)
  cp = pltpu.CompilerParams(needs_layout_passes=False)

  run_ab = pl.kernel(
      _body_ab,
      out_type=(
          jax.ShapeDtypeStruct((NW * CAP,), jnp.int32),
          jax.ShapeDtypeStruct((NW * 16,), jnp.int32),
          jax.ShapeDtypeStruct((NW * CAP, D), jnp.float32),
      ),
      mesh=mesh,
      compiler_params=cp,
      scratch_types=[
          pltpu.VMEM((B,), jnp.int32),
          pltpu.VMEM((R,), jnp.int32),
          pltpu.VMEM((CAP + 16,), jnp.int32),
          pltpu.VMEM((CAP + 16,), jnp.int32),
          pltpu.VMEM((16,), jnp.int32),
          pltpu.VMEM((C, D), jnp.float32),
          pltpu.VMEM((C, D), jnp.float32),
          pltpu.VMEM((C, D), jnp.float32),
          pltpu.VMEM((C, D), jnp.float32),
          pltpu.SemaphoreType.DMA,
          pltpu.SemaphoreType.DMA,
          pltpu.SemaphoreType.DMA,
          pltpu.SemaphoreType.DMA,
      ],
  )
  dstl_hbm, nch_hbm, norm_hbm = run_ab(p_labels, f_out, features)

  copied = pl.pallas_call(
      _copy_block,
      grid=(N_ROWS // _COPY_BLK,),
      in_specs=[pl.BlockSpec((_COPY_BLK, D), lambda i: (i, 0))],
      out_specs=pl.BlockSpec((_COPY_BLK, D), lambda i: (i, 0)),
      out_shape=jax.ShapeDtypeStruct((N_ROWS, D), jnp.float32),
  )(features)
  out_ref = jax.new_ref(copied)
  run_b2 = pl.kernel(
      _body_b2,
      out_type=(),
      mesh=mesh,
      compiler_params=cp,
      scratch_types=[
          pltpu.VMEM((CAP + 16,), jnp.int32),
          pltpu.VMEM((NCH, 1, C), jnp.int32),
          pltpu.VMEM((16,), jnp.int32),
          pltpu.VMEM((C, D), jnp.float32),
          pltpu.VMEM((C, D), jnp.float32),
          pltpu.SemaphoreType.DMA,
          pltpu.SemaphoreType.DMA,
          pltpu.SemaphoreType.DMA,
          pltpu.SemaphoreType.DMA,
      ],
  )
  run_b2(dstl_hbm, nch_hbm, norm_hbm, out_ref)
  return out_ref[...]


# speedup vs baseline: 25.3206x; 1.0043x over previous
import jax
import jax.numpy as jnp
from jax import lax
from jax.experimental import pallas as pl
from jax.experimental.pallas import tpu as pltpu, tpu_sc as plsc

N_ROWS = 100000
D = 256
B = 16384
MOM = 0.2

NC = 2
NS = 16
NW = NC * NS
R = 3136
C = 96
CAP = ((R + C - 1) // C) * C
NCH = CAP // C
DV = D // 16

_SENT = 0x7FFFFFFF


def _take(v, idx):
  return jnp.take_along_axis(v, idx, axis=0)


def _body_ab(plab_hbm, fout_hbm, feat_hbm, dstl_hbm, nch_hbm, norm_hbm,
             labels_v, claim, srcs, dstl, nch_v, fbuf0, gbuf0, fbuf1, gbuf1,
             sem_g0, sem_g1, sem_s0, sem_s1):
  wid = lax.axis_index("s") * NC + lax.axis_index("c")
  lo = wid * R
  hi = lo + R
  iota = lax.iota(jnp.int32, 16)
  nxt_idx = (iota + 1) & 15

  pltpu.sync_copy(plab_hbm, labels_v)

  minus1 = jnp.full((16,), -1, jnp.int32)

  @pl.loop(0, R // 16)
  def _(k):
    claim[pl.ds(k * 16, 16)] = minus1

  @pl.loop(0, B // 16, unroll=2, init_carry=jnp.int32(0))
  def pass1(s, cnt):
    l = labels_v[pl.ds(s * 16, 16)]
    i = s * 16 + iota
    inr = (l >= lo) & (l < hi)
    comp = ((l - lo) << 14) | i
    mi = jnp.where(inr, jnp.int32(1), jnp.int32(0))
    cum = plsc.cumsum(mi)
    posw = jnp.where(inr, cnt + cum - 1, 0)
    plsc.store_scatter(labels_v, (posw,), comp, mask=inr)
    return cnt + plsc.all_reduce_population_count(inr)[0]

  k2 = pass1

  @pl.loop(0, (k2 + 15) // 16)
  def _(q):
    offs = q * 16 + iota
    mk = offs < k2
    cv = labels_v[pl.ds(q * 16, 16)]
    comp = jnp.where(mk, cv, _SENT)
    sk, _ = plsc.sort_key_val(comp, comp)
    slab = sk >> 14
    nlab = _take(slab, nxt_idx)
    win = ((slab != nlab) | (iota == 15)) & (sk != _SENT)
    idx = jnp.where(win, slab, 0)
    plsc.store_scatter(claim, (idx,), sk & 0x3FFF, mask=win)

  @pl.loop(0, R // 16, init_carry=jnp.int32(0))
  def count(k, cnt):
    c = claim[pl.ds(k * 16, 16)]
    m = c >= 0
    mi = jnp.where(m, jnp.int32(1), jnp.int32(0))
    cum = plsc.cumsum(mi)
    posw = jnp.where(m, cnt + cum - 1, 0)
    plsc.store_scatter(srcs, (posw,), c, mask=m)
    plsc.store_scatter(dstl, (posw,), lo + k * 16 + iota, mask=m)
    return cnt + jnp.sum(mi)

  k_cnt = count

  @pl.when(k_cnt > 0)
  def _():
    kpad = ((k_cnt + C - 1) // C) * C

    @pl.loop(0, C // 16)
    def _(j):
      offs = k_cnt + j * 16 + iota
      mk = offs < kpad
      offw = jnp.where(mk, offs, 0)
      srcoff = jnp.maximum(offw - C, 0)
      plsc.store_scatter(srcs, (offw,), plsc.load_gather(srcs, (srcoff,)),
                         mask=mk)
      plsc.store_scatter(dstl, (offw,), plsc.load_gather(dstl, (srcoff,)),
                         mask=mk)

  nchunks = (k_cnt + C - 1) // C
  nch_v[pl.ds(0, 16)] = jnp.full((16,), 0, jnp.int32) + nchunks

  pltpu.sync_copy(dstl.at[pl.ds(0, CAP)], dstl_hbm.at[pl.ds(wid * CAP, CAP)])
  pltpu.sync_copy(nch_v, nch_hbm.at[pl.ds(wid * 16, 16)])

  bufs = ((fbuf0, gbuf0, sem_g0, sem_s0), (fbuf1, gbuf1, sem_g1, sem_s1))

  def start_gather(t, fb, gb, sg):
    pltpu.make_async_copy(
        fout_hbm.at[srcs.at[pl.ds(t * C, C)]], fb, sg).start()
    pltpu.make_async_copy(
        feat_hbm.at[dstl.at[pl.ds(t * C, C)]], gb, sg).start()

  def wait_gather(t, fb, gb, sg):
    pltpu.make_async_copy(
        fout_hbm.at[srcs.at[pl.ds(t * C, C)]], fb, sg).wait()
    pltpu.make_async_copy(
        feat_hbm.at[dstl.at[pl.ds(t * C, C)]], gb, sg).wait()

  def norm_slice(t):
    return norm_hbm.at[pl.ds(wid * CAP + t * C, C)]

  @pl.when(nchunks > 0)
  def _():
    start_gather(0, fbuf0, gbuf0, sem_g0)

  @pl.loop(0, nchunks)
  def _(t):
    par = t & 1

    for p in range(2):
      fb, gb, sg, ss = bufs[p]

      @pl.when(par == p)
      def _():
        @pl.when(t >= 1)
        def _():
          ofb = bufs[1 - p][0]
          oss = bufs[1 - p][3]
          pltpu.make_async_copy(ofb, norm_slice(t - 1), oss).wait()

        @pl.when(t + 1 < nchunks)
        def _():
          nfb, ngb, nsg, _ = bufs[1 - p]
          start_gather(t + 1, nfb, ngb, nsg)

        wait_gather(t, fb, gb, sg)

        @pl.loop(0, C, unroll=2)
        def _(r):
          acc = jnp.zeros((16,), jnp.float32)
          m = []
          for j in range(DV):
            g = gb[r, pl.ds(j * 16, 16)]
            f = fb[r, pl.ds(j * 16, 16)]
            mj = MOM * g + (1.0 - MOM) * f
            m.append(mj)
            acc = acc + mj * mj
          tot = _take(plsc.cumsum(acc), jnp.full((16,), 15, jnp.int32))
          bits = plsc.bitcast(tot, jnp.int32)
          y = plsc.bitcast(jnp.int32(0x5F3759DF) - (bits >> 1), jnp.float32)
          for _ in range(2):
            y = y * (1.5 - 0.5 * tot * y * y)
          for j in range(DV):
            fb[r, pl.ds(j * 16, 16)] = m[j] * y

        pltpu.make_async_copy(fb, norm_slice(t), ss).start()

  @pl.when(nchunks > 0)
  def _():
    for p in range(2):
      fb, _, _, ss = bufs[p]

      @pl.when((nchunks - 1) & 1 == p)
      def _():
        pltpu.make_async_copy(fb, norm_slice(nchunks - 1), ss).wait()


def _body_b2(dstl_hbm, nch_hbm, norm_hbm, out_ref,
             dstl, dstl3d, nch_v, buf0, buf1,
             sem_l0, sem_l1, sem_s0, sem_s1):
  wid = lax.axis_index("s") * NC + lax.axis_index("c")

  pltpu.sync_copy(dstl_hbm.at[pl.ds(wid * CAP, CAP)], dstl.at[pl.ds(0, CAP)])
  pltpu.sync_copy(nch_hbm.at[pl.ds(wid * 16, 16)], nch_v)
  nchunks = nch_v[pl.ds(0, 16)][0]

  @pl.loop(0, CAP // 16)
  def _(k):
    v = dstl[pl.ds(k * 16, 16)]
    ch = k // (C // 16)
    off = (k - ch * (C // 16)) * 16
    dstl3d[ch, 0, pl.ds(off, 16)] = v

  bufs = ((buf0, sem_l0, sem_s0), (buf1, sem_l1, sem_s1))

  def norm_slice(t):
    return norm_hbm.at[pl.ds(wid * CAP + t * C, C)]

  @pl.when(nchunks > 0)
  def _():
    pltpu.make_async_copy(norm_slice(0), buf0, sem_l0).start()

  @pl.loop(0, nchunks)
  def _(t):
    par = t & 1

    for p in range(2):
      bf, sl, ss = bufs[p]

      @pl.when(par == p)
      def _():
        @pl.when(t >= 1)
        def _():
          obf, _, oss = bufs[1 - p]
          pltpu.make_async_copy(
              obf, out_ref.at[dstl3d.at[t - 1, 0]], oss).wait()

        @pl.when(t + 1 < nchunks)
        def _():
          nbf, nsl, _ = bufs[1 - p]
          pltpu.make_async_copy(norm_slice(t + 1), nbf, nsl).start()

        pltpu.make_async_copy(norm_slice(t), bf, sl).wait()
        pltpu.make_async_copy(bf, out_ref.at[dstl3d.at[t, 0]], ss).start()

  @pl.when(nchunks > 0)
  def _():
    for p in range(2):
      bf, _, ss = bufs[p]

      @pl.when((nchunks - 1) & 1 == p)
      def _():
        pltpu.make_async_copy(
            bf, out_ref.at[dstl3d.at[nchunks - 1, 0]], ss).wait()


_COPY_BLK = 2000


def _copy_block(x_ref, o_ref):
  o_ref[...] = x_ref[...]


def kernel(f_out, p_labels, features):
  mesh = plsc.VectorSubcoreMesh(
      core_axis_name="c", subcore_axis_name="s", num_cores=NC)
  cp = pltpu.CompilerParams(needs_layout_passes=False)

  run_ab = pl.kernel(
      _body_ab,
      out_type=(
          jax.ShapeDtypeStruct((NW * CAP,), jnp.int32),
          jax.ShapeDtypeStruct((NW * 16,), jnp.int32),
          jax.ShapeDtypeStruct((NW * CAP, D), jnp.float32),
      ),
      mesh=mesh,
      compiler_params=cp,
      scratch_types=[
          pltpu.VMEM((B,), jnp.int32),
          pltpu.VMEM((R,), jnp.int32),
          pltpu.VMEM((CAP + 16,), jnp.int32),
          pltpu.VMEM((CAP + 16,), jnp.int32),
          pltpu.VMEM((16,), jnp.int32),
          pltpu.VMEM((C, D), jnp.float32),
          pltpu.VMEM((C, D), jnp.float32),
          pltpu.VMEM((C, D), jnp.float32),
          pltpu.VMEM((C, D), jnp.float32),
          pltpu.SemaphoreType.DMA,
          pltpu.SemaphoreType.DMA,
          pltpu.SemaphoreType.DMA,
          pltpu.SemaphoreType.DMA,
      ],
  )
  dstl_hbm, nch_hbm, norm_hbm = run_ab(p_labels, f_out, features)

  copied = pl.pallas_call(
      _copy_block,
      grid=(N_ROWS // _COPY_BLK,),
      in_specs=[pl.BlockSpec((_COPY_BLK, D), lambda i: (i, 0))],
      out_specs=pl.BlockSpec((_COPY_BLK, D), lambda i: (i, 0)),
      out_shape=jax.ShapeDtypeStruct((N_ROWS, D), jnp.float32),
  )(features)
  out_ref = jax.new_ref(copied)
  run_b2 = pl.kernel(
      _body_b2,
      out_type=(),
      mesh=mesh,
      compiler_params=cp,
      scratch_types=[
          pltpu.VMEM((CAP + 16,), jnp.int32),
          pltpu.VMEM((NCH, 1, C), jnp.int32),
          pltpu.VMEM((16,), jnp.int32),
          pltpu.VMEM((C, D), jnp.float32),
          pltpu.VMEM((C, D), jnp.float32),
          pltpu.SemaphoreType.DMA,
          pltpu.SemaphoreType.DMA,
          pltpu.SemaphoreType.DMA,
          pltpu.SemaphoreType.DMA,
      ],
  )
  run_b2(dstl_hbm, nch_hbm, norm_hbm, out_ref)
  return out_ref[...]


# speedup vs baseline: 26.8516x; 1.0605x over previous
import jax
import jax.numpy as jnp
from jax import lax
from jax.experimental import pallas as pl
from jax.experimental.pallas import tpu as pltpu, tpu_sc as plsc

N_ROWS = 100000
D = 256
B = 16384
MOM = 0.2

NC = 2
NS = 16
NW = NC * NS
R = 3136
R16 = R
C = 112
CAP = ((R + C - 1) // C) * C
NCH = CAP // C
DV = D // 16

_SENT = 0x7FFFFFFF


def _take(v, idx):
  return jnp.take_along_axis(v, idx, axis=0)


def _splat0(v16):
  return _take(v16, jnp.zeros((16,), jnp.int32))


def _body_a(plab_hbm, srcs_hbm, dstl_hbm, nch_hbm, labels_v, claim, srcs,
            dstl, nch_v, sem):
  wid = lax.axis_index("s") * NC + lax.axis_index("c")
  lo = wid * R
  hi = lo + R
  iota = lax.iota(jnp.int32, 16)
  nxt_idx = (iota + 1) & 15

  pltpu.sync_copy(plab_hbm, labels_v)

  minus1 = jnp.full((16,), -1, jnp.int32)

  @pl.loop(0, R16 // 16)
  def _(k):
    claim[pl.ds(k * 16, 16)] = minus1

  @pl.loop(0, B // 16, unroll=2, init_carry=jnp.int32(0))
  def pass1(s, cnt):
    l = labels_v[pl.ds(s * 16, 16)]
    i = s * 16 + iota
    inr = (l >= lo) & (l < hi)
    comp = ((l - lo) << 14) | i
    mi = jnp.where(inr, jnp.int32(1), jnp.int32(0))
    cum = plsc.cumsum(mi)
    posw = jnp.where(inr, cnt + cum - 1, 0)
    plsc.store_scatter(labels_v, (posw,), comp, mask=inr)
    return cnt + plsc.all_reduce_population_count(inr)[0]

  k2 = pass1

  @pl.loop(0, (k2 + 15) // 16)
  def _(q):
    offs = q * 16 + iota
    mk = offs < k2
    cv = labels_v[pl.ds(q * 16, 16)]
    comp = jnp.where(mk, cv, _SENT)
    sk, _ = plsc.sort_key_val(comp, comp)
    slab = sk >> 14
    nlab = _take(slab, nxt_idx)
    win = ((slab != nlab) | (iota == 15)) & (sk != _SENT)
    idx = jnp.where(win, slab, 0)
    plsc.store_scatter(claim, (idx,), sk & 0x3FFF, mask=win)

  @pl.loop(0, R16 // 16, init_carry=jnp.int32(0))
  def count(k, cnt):
    c = claim[pl.ds(k * 16, 16)]
    m = c >= 0
    mi = jnp.where(m, jnp.int32(1), jnp.int32(0))
    cum = plsc.cumsum(mi)
    posw = jnp.where(m, cnt + cum - 1, 0)
    plsc.store_scatter(srcs, (posw,), c, mask=m)
    plsc.store_scatter(dstl, (posw,), lo + k * 16 + iota, mask=m)
    return cnt + jnp.sum(mi)

  k_cnt = count

  @pl.when(k_cnt > 0)
  def _():
    kpad = ((k_cnt + C - 1) // C) * C

    @pl.loop(0, C // 16)
    def _(j):
      offs = k_cnt + j * 16 + iota
      mk = offs < kpad
      offw = jnp.where(mk, offs, 0)
      srcoff = jnp.maximum(offw - C, 0)
      plsc.store_scatter(srcs, (offw,), plsc.load_gather(srcs, (srcoff,)),
                         mask=mk)
      plsc.store_scatter(dstl, (offw,), plsc.load_gather(dstl, (srcoff,)),
                         mask=mk)

  nchunks = (k_cnt + C - 1) // C
  nch_v[pl.ds(0, 16)] = jnp.full((16,), 0, jnp.int32) + nchunks

  pltpu.sync_copy(srcs.at[pl.ds(0, CAP)], srcs_hbm.at[pl.ds(wid * CAP, CAP)])
  pltpu.sync_copy(dstl.at[pl.ds(0, CAP)], dstl_hbm.at[pl.ds(wid * CAP, CAP)])
  pltpu.sync_copy(nch_v, nch_hbm.at[pl.ds(wid * 16, 16)])


def _body_b(srcs_hbm, dstl_hbm, nch_hbm, fout_hbm, feat_hbm, out_ref,
            srcs, dstl, dstl3d, nch_v, fbuf0, gbuf0, fbuf1, gbuf1,
            sem_g0, sem_g1, sem_s0, sem_s1):
  wid = lax.axis_index("s") * NC + lax.axis_index("c")

  pltpu.sync_copy(srcs_hbm.at[pl.ds(wid * CAP, CAP)], srcs.at[pl.ds(0, CAP)])
  pltpu.sync_copy(dstl_hbm.at[pl.ds(wid * CAP, CAP)], dstl.at[pl.ds(0, CAP)])
  pltpu.sync_copy(nch_hbm.at[pl.ds(wid * 16, 16)], nch_v)
  nchunks = nch_v[pl.ds(0, 16)][0]

  @pl.loop(0, CAP // 16)
  def _(k):
    v = dstl[pl.ds(k * 16, 16)]
    ch = k // (C // 16)
    off = (k - ch * (C // 16)) * 16
    dstl3d[ch, 0, pl.ds(off, 16)] = v

  bufs = ((fbuf0, gbuf0, sem_g0, sem_s0), (fbuf1, gbuf1, sem_g1, sem_s1))

  def start_gather(t, fb, gb, sg):
    pltpu.make_async_copy(
        fout_hbm.at[srcs.at[pl.ds(t * C, C)]], fb, sg).start()
    pltpu.make_async_copy(feat_hbm.at[dstl3d.at[t, 0]], gb, sg).start()

  def wait_gather(t, fb, gb, sg):
    pltpu.make_async_copy(
        fout_hbm.at[srcs.at[pl.ds(t * C, C)]], fb, sg).wait()
    pltpu.make_async_copy(feat_hbm.at[dstl3d.at[t, 0]], gb, sg).wait()

  @pl.when(nchunks > 0)
  def _():
    start_gather(0, fbuf0, gbuf0, sem_g0)

  @pl.loop(0, nchunks)
  def _(t):
    par = t & 1

    for p in range(2):
      fb, gb, sg, ss = bufs[p]

      @pl.when(par == p)
      def _():
        @pl.when(t >= 1)
        def _():
          ofb, _, _, oss = bufs[1 - p]
          pltpu.make_async_copy(
              ofb, out_ref.at[dstl3d.at[t - 1, 0]], oss).wait()

        @pl.when(t + 1 < nchunks)
        def _():
          nfb, ngb, nsg, _ = bufs[1 - p]
          start_gather(t + 1, nfb, ngb, nsg)

        wait_gather(t, fb, gb, sg)

        @pl.loop(0, C, unroll=2)
        def _(r):
          acc = jnp.zeros((16,), jnp.float32)
          m = []
          for j in range(DV):
            g = gb[r, pl.ds(j * 16, 16)]
            f = fb[r, pl.ds(j * 16, 16)]
            mj = MOM * g + (1.0 - MOM) * f
            m.append(mj)
            acc = acc + mj * mj
          tot = _take(plsc.cumsum(acc), jnp.full((16,), 15, jnp.int32))
          bits = plsc.bitcast(tot, jnp.int32)
          y = plsc.bitcast(jnp.int32(0x5F3759DF) - (bits >> 1), jnp.float32)
          for _ in range(2):
            y = y * (1.5 - 0.5 * tot * y * y)
          for j in range(DV):
            fb[r, pl.ds(j * 16, 16)] = m[j] * y

        pltpu.make_async_copy(fb, out_ref.at[dstl3d.at[t, 0]], ss).start()

  @pl.when(nchunks > 0)
  def _():
    for p in range(2):
      fb, _, _, ss = bufs[p]

      @pl.when((nchunks - 1) & 1 == p)
      def _():
        pltpu.make_async_copy(
            fb, out_ref.at[dstl3d.at[nchunks - 1, 0]], ss).wait()


def kernel(f_out, p_labels, features):
  mesh = plsc.VectorSubcoreMesh(
      core_axis_name="c", subcore_axis_name="s", num_cores=NC)
  cp = pltpu.CompilerParams(needs_layout_passes=False)

  run_a = pl.kernel(
      _body_a,
      out_type=(
          jax.ShapeDtypeStruct((NW * CAP,), jnp.int32),
          jax.ShapeDtypeStruct((NW * CAP,), jnp.int32),
          jax.ShapeDtypeStruct((NW * 16,), jnp.int32),
      ),
      mesh=mesh,
      compiler_params=cp,
      scratch_types=[
          pltpu.VMEM((B,), jnp.int32),
          pltpu.VMEM((R16,), jnp.int32),
          pltpu.VMEM((CAP + 16,), jnp.int32),
          pltpu.VMEM((CAP + 16,), jnp.int32),
          pltpu.VMEM((16,), jnp.int32),
          pltpu.SemaphoreType.DMA,
      ],
  )
  srcs_hbm, dstl_hbm, nch_hbm = run_a(p_labels)

  out_ref = jax.new_ref(features)
  run_b = pl.kernel(
      _body_b,
      out_type=(),
      mesh=mesh,
      compiler_params=cp,
      scratch_types=[
          pltpu.VMEM((CAP + 16,), jnp.int32),
          pltpu.VMEM((CAP + 16,), jnp.int32),
          pltpu.VMEM((NCH, 1, C), jnp.int32),
          pltpu.VMEM((16,), jnp.int32),
          pltpu.VMEM((C, D), jnp.float32),
          pltpu.VMEM((C, D), jnp.float32),
          pltpu.VMEM((C, D), jnp.float32),
          pltpu.VMEM((C, D), jnp.float32),
          pltpu.SemaphoreType.DMA,
          pltpu.SemaphoreType.DMA,
          pltpu.SemaphoreType.DMA,
          pltpu.SemaphoreType.DMA,
      ],
  )
  run_b(srcs_hbm, dstl_hbm, nch_hbm, f_out, features, out_ref)
  return out_ref[...]


# speedup vs baseline: 27.8009x; 1.0354x over previous
import jax
import jax.numpy as jnp
from jax import lax
from jax.experimental import pallas as pl
from jax.experimental.pallas import tpu as pltpu, tpu_sc as plsc

N_ROWS = 100000
D = 256
B = 16384
MOM = 0.2

NC = 2
NS = 16
NW = NC * NS
R = 3136
R16 = R
C = 48
NBUF = 4
CAP = ((R + C - 1) // C) * C
NCH = CAP // C
DV = D // 16

_SENT = 0x7FFFFFFF


def _take(v, idx):
  return jnp.take_along_axis(v, idx, axis=0)


def _splat0(v16):
  return _take(v16, jnp.zeros((16,), jnp.int32))


def _body_a(plab_hbm, srcs_hbm, dstl_hbm, nch_hbm, labels_v, claim, srcs,
            dstl, nch_v, sem):
  wid = lax.axis_index("s") * NC + lax.axis_index("c")
  lo = wid * R
  hi = lo + R
  iota = lax.iota(jnp.int32, 16)
  nxt_idx = (iota + 1) & 15

  pltpu.sync_copy(plab_hbm, labels_v)

  minus1 = jnp.full((16,), -1, jnp.int32)

  @pl.loop(0, R16 // 16)
  def _(k):
    claim[pl.ds(k * 16, 16)] = minus1

  @pl.loop(0, B // 16, unroll=2, init_carry=jnp.int32(0))
  def pass1(s, cnt):
    l = labels_v[pl.ds(s * 16, 16)]
    i = s * 16 + iota
    inr = (l >= lo) & (l < hi)
    comp = ((l - lo) << 14) | i
    mi = jnp.where(inr, jnp.int32(1), jnp.int32(0))
    cum = plsc.cumsum(mi)
    posw = jnp.where(inr, cnt + cum - 1, 0)
    plsc.store_scatter(labels_v, (posw,), comp, mask=inr)
    return cnt + plsc.all_reduce_population_count(inr)[0]

  k2 = pass1

  @pl.loop(0, (k2 + 15) // 16)
  def _(q):
    offs = q * 16 + iota
    mk = offs < k2
    cv = labels_v[pl.ds(q * 16, 16)]
    comp = jnp.where(mk, cv, _SENT)
    sk, _ = plsc.sort_key_val(comp, comp)
    slab = sk >> 14
    nlab = _take(slab, nxt_idx)
    win = ((slab != nlab) | (iota == 15)) & (sk != _SENT)
    idx = jnp.where(win, slab, 0)
    plsc.store_scatter(claim, (idx,), sk & 0x3FFF, mask=win)

  @pl.loop(0, R16 // 16, init_carry=jnp.int32(0))
  def count(k, cnt):
    c = claim[pl.ds(k * 16, 16)]
    m = c >= 0
    mi = jnp.where(m, jnp.int32(1), jnp.int32(0))
    cum = plsc.cumsum(mi)
    posw = jnp.where(m, cnt + cum - 1, 0)
    plsc.store_scatter(srcs, (posw,), c, mask=m)
    plsc.store_scatter(dstl, (posw,), lo + k * 16 + iota, mask=m)
    return cnt + jnp.sum(mi)

  k_cnt = count

  @pl.when(k_cnt > 0)
  def _():
    kpad = ((k_cnt + C - 1) // C) * C

    @pl.loop(0, C // 16)
    def _(j):
      offs = k_cnt + j * 16 + iota
      mk = offs < kpad
      offw = jnp.where(mk, offs, 0)
      srcoff = jnp.maximum(offw - C, 0)
      plsc.store_scatter(srcs, (offw,), plsc.load_gather(srcs, (srcoff,)),
                         mask=mk)
      plsc.store_scatter(dstl, (offw,), plsc.load_gather(dstl, (srcoff,)),
                         mask=mk)

  nchunks = (k_cnt + C - 1) // C
  nch_v[pl.ds(0, 16)] = jnp.full((16,), 0, jnp.int32) + nchunks

  pltpu.sync_copy(srcs.at[pl.ds(0, CAP)], srcs_hbm.at[pl.ds(wid * CAP, CAP)])
  pltpu.sync_copy(dstl.at[pl.ds(0, CAP)], dstl_hbm.at[pl.ds(wid * CAP, CAP)])
  pltpu.sync_copy(nch_v, nch_hbm.at[pl.ds(wid * 16, 16)])


def _body_b(srcs_hbm, dstl_hbm, nch_hbm, fout_hbm, feat_hbm, out_ref,
            srcs, dstl, dstl3d, nch_v, fbuf0, gbuf0, fbuf1, gbuf1,
            fbuf2, gbuf2, fbuf3, gbuf3,
            sem_g0, sem_g1, sem_g2, sem_g3, sem_s0, sem_s1, sem_s2, sem_s3):
  wid = lax.axis_index("s") * NC + lax.axis_index("c")

  pltpu.sync_copy(srcs_hbm.at[pl.ds(wid * CAP, CAP)], srcs.at[pl.ds(0, CAP)])
  pltpu.sync_copy(dstl_hbm.at[pl.ds(wid * CAP, CAP)], dstl.at[pl.ds(0, CAP)])
  pltpu.sync_copy(nch_hbm.at[pl.ds(wid * 16, 16)], nch_v)
  nchunks = nch_v[pl.ds(0, 16)][0]

  @pl.loop(0, CAP // 16)
  def _(k):
    v = dstl[pl.ds(k * 16, 16)]
    ch = k // (C // 16)
    off = (k - ch * (C // 16)) * 16
    dstl3d[ch, 0, pl.ds(off, 16)] = v

  bufs = ((fbuf0, gbuf0, sem_g0, sem_s0), (fbuf1, gbuf1, sem_g1, sem_s1),
          (fbuf2, gbuf2, sem_g2, sem_s2), (fbuf3, gbuf3, sem_g3, sem_s3))

  def start_gather(t, fb, gb, sg):
    pltpu.make_async_copy(
        fout_hbm.at[srcs.at[pl.ds(t * C, C)]], fb, sg).start()
    pltpu.make_async_copy(feat_hbm.at[dstl3d.at[t, 0]], gb, sg).start()

  def wait_gather(t, fb, gb, sg):
    pltpu.make_async_copy(
        fout_hbm.at[srcs.at[pl.ds(t * C, C)]], fb, sg).wait()
    pltpu.make_async_copy(feat_hbm.at[dstl3d.at[t, 0]], gb, sg).wait()

  @pl.when(nchunks > 0)
  def _():
    start_gather(0, fbuf0, gbuf0, sem_g0)

  @pl.when(nchunks > 1)
  def _():
    start_gather(1, fbuf1, gbuf1, sem_g1)

  @pl.loop(0, nchunks)
  def _(t):
    par = t & 3

    for p in range(NBUF):
      fb, gb, sg, ss = bufs[p]
      p2 = (p + 2) & 3

      @pl.when(par == p)
      def _():
        @pl.when(t >= 2)
        def _():
          ofb = bufs[p2][0]
          oss = bufs[p2][3]
          pltpu.make_async_copy(
              ofb, out_ref.at[dstl3d.at[t - 2, 0]], oss).wait()

        @pl.when(t + 2 < nchunks)
        def _():
          nfb, ngb, nsg, _ = bufs[p2]
          start_gather(t + 2, nfb, ngb, nsg)

        wait_gather(t, fb, gb, sg)

        @pl.loop(0, C, unroll=2)
        def _(r):
          acc = jnp.zeros((16,), jnp.float32)
          m = []
          for j in range(DV):
            g = gb[r, pl.ds(j * 16, 16)]
            f = fb[r, pl.ds(j * 16, 16)]
            mj = MOM * g + (1.0 - MOM) * f
            m.append(mj)
            acc = acc + mj * mj
          tot = _take(plsc.cumsum(acc), jnp.full((16,), 15, jnp.int32))
          bits = plsc.bitcast(tot, jnp.int32)
          y = plsc.bitcast(jnp.int32(0x5F3759DF) - (bits >> 1), jnp.float32)
          for _ in range(2):
            y = y * (1.5 - 0.5 * tot * y * y)
          for j in range(DV):
            fb[r, pl.ds(j * 16, 16)] = m[j] * y

        pltpu.make_async_copy(fb, out_ref.at[dstl3d.at[t, 0]], ss).start()

  for back in (1, 2):
    @pl.when(nchunks >= back)
    def _():
      for p in range(NBUF):
        fb, _, _, ss = bufs[p]

        @pl.when((nchunks - back) & 3 == p)
        def _():
          pltpu.make_async_copy(
              fb, out_ref.at[dstl3d.at[nchunks - back, 0]], ss).wait()


def kernel(f_out, p_labels, features):
  mesh = plsc.VectorSubcoreMesh(
      core_axis_name="c", subcore_axis_name="s", num_cores=NC)
  cp = pltpu.CompilerParams(needs_layout_passes=False)

  run_a = pl.kernel(
      _body_a,
      out_type=(
          jax.ShapeDtypeStruct((NW * CAP,), jnp.int32),
          jax.ShapeDtypeStruct((NW * CAP,), jnp.int32),
          jax.ShapeDtypeStruct((NW * 16,), jnp.int32),
      ),
      mesh=mesh,
      compiler_params=cp,
      scratch_types=[
          pltpu.VMEM((B,), jnp.int32),
          pltpu.VMEM((R16,), jnp.int32),
          pltpu.VMEM((CAP + 16,), jnp.int32),
          pltpu.VMEM((CAP + 16,), jnp.int32),
          pltpu.VMEM((16,), jnp.int32),
          pltpu.SemaphoreType.DMA,
      ],
  )
  srcs_hbm, dstl_hbm, nch_hbm = run_a(p_labels)

  out_ref = jax.new_ref(features)
  run_b = pl.kernel(
      _body_b,
      out_type=(),
      mesh=mesh,
      compiler_params=cp,
      scratch_types=[
          pltpu.VMEM((CAP + 16,), jnp.int32),
          pltpu.VMEM((CAP + 16,), jnp.int32),
          pltpu.VMEM((NCH, 1, C), jnp.int32),
          pltpu.VMEM((16,), jnp.int32),
      ] + [pltpu.VMEM((C, D), jnp.float32)] * 8
        + [pltpu.SemaphoreType.DMA] * 8,
  )
  run_b(srcs_hbm, dstl_hbm, nch_hbm, f_out, features, out_ref)
  return out_ref[...]
